# baseline (speedup 1.0000x reference)
import functools

import jax
import jax.numpy as jnp
from jax import lax
from jax.experimental import pallas as pl
from jax.experimental.pallas import tpu as pltpu
from jax.experimental.pallas import tpu_sc as plsc

N = 8192
D = 2048
E = 8
K = 2
TM = 256
NT = (N * K) // TM + E
SPAD = NT * TM
NW = 32
SLOTS_W = SPAD // NW
SLOTS_WP = 640
TOK_W = N // NW
GROWS = 32
CROWS = 16
RBLK = 128
NRB = N // RBLK

_CLAMP = float(jnp.log(jnp.asarray(100.0, dtype=jnp.float32)))


def _router_body(temp_ref, x_ref, mem_ref, i0_ref, i1_ref, w0_ref, w1_ref,
                 load_ref, vqs_ref):
    i = pl.program_id(0)
    x = x_ref[...]
    mem = mem_ref[...]
    memn = mem / jnp.maximum(
        jnp.sqrt(jnp.sum(mem * mem, axis=1, keepdims=True)), 1e-12)
    xn = x / jnp.maximum(
        jnp.sqrt(jnp.sum(x * x, axis=1, keepdims=True)), 1e-12)
    dists = lax.dot_general(memn, xn, (((1,), (1,)), ((), ())),
                            preferred_element_type=jnp.float32)
    e_iota = lax.broadcasted_iota(jnp.int32, (E, RBLK), 0)
    v0 = jnp.max(dists, axis=0, keepdims=True)
    i0 = jnp.min(jnp.where(dists >= v0, e_iota, E), axis=0, keepdims=True)
    masked = jnp.where(e_iota == i0, -2.0, dists)
    v1 = jnp.max(masked, axis=0, keepdims=True)
    i1 = jnp.min(jnp.where(masked >= v1, e_iota, E), axis=0, keepdims=True)
    e1 = jnp.exp(v1 - v0)
    w0 = 1.0 / (1.0 + e1)
    w1 = e1 / (1.0 + e1)
    scale = jnp.exp(jnp.minimum(temp_ref[0], _CLAMP))
    i0_ref[...] = i0.reshape(1, 1, RBLK)
    i1_ref[...] = i1.reshape(1, 1, RBLK)
    w0_ref[...] = (scale * w0).reshape(1, 1, RBLK)
    w1_ref[...] = (scale * w1).reshape(1, 1, RBLK)
    aw = (jnp.where(e_iota == i0, w0, 0.0)
          + jnp.where(e_iota == i1, w1, 0.0))
    load_part = jnp.sum(aw, axis=1, keepdims=True)
    vq_part = jnp.sum(w0 * v0 + w1 * v1)

    @pl.when(i == 0)
    def _():
        load_ref[...] = jnp.zeros((E, 128), jnp.float32)
        vqs_ref[...] = jnp.zeros((E, 128), jnp.float32)

    load_ref[...] += jnp.broadcast_to(load_part, (E, 128))
    vqs_ref[...] += jnp.full((E, 128), vq_part)


def _router(x, routing_memory, temperature):
    return pl.pallas_call(
        _router_body,
        grid=(NRB,),
        in_specs=[
            pl.BlockSpec(memory_space=pltpu.SMEM),
            pl.BlockSpec((RBLK, D), lambda i: (i, 0)),
            pl.BlockSpec((E, D), lambda i: (0, 0)),
        ],
        out_specs=[
            pl.BlockSpec((1, 1, RBLK), lambda i: (i, 0, 0)),
            pl.BlockSpec((1, 1, RBLK), lambda i: (i, 0, 0)),
            pl.BlockSpec((1, 1, RBLK), lambda i: (i, 0, 0)),
            pl.BlockSpec((1, 1, RBLK), lambda i: (i, 0, 0)),
            pl.BlockSpec((E, 128), lambda i: (0, 0)),
            pl.BlockSpec((E, 128), lambda i: (0, 0)),
        ],
        out_shape=[
            jax.ShapeDtypeStruct((NRB, 1, RBLK), jnp.int32),
            jax.ShapeDtypeStruct((NRB, 1, RBLK), jnp.int32),
            jax.ShapeDtypeStruct((NRB, 1, RBLK), jnp.float32),
            jax.ShapeDtypeStruct((NRB, 1, RBLK), jnp.float32),
            jax.ShapeDtypeStruct((E, 128), jnp.float32),
            jax.ShapeDtypeStruct((E, 128), jnp.float32),
        ],
    )(temperature, x, routing_memory)


def _dispatch_body(i0_ref, i1_ref, load_ref, vqs_ref, mem_ref,
                   d0_ref, d1_ref, eot_ref, aux_ref):
    i0 = i0_ref[...].reshape(NRB, RBLK)
    i1 = i1_ref[...].reshape(NRB, RBLK)
    r128 = lax.broadcasted_iota(jnp.int32, (RBLK, RBLK), 0)
    c128 = lax.broadcasted_iota(jnp.int32, (RBLK, RBLK), 1)
    U = (r128 <= c128).astype(jnp.float32)
    r64 = lax.broadcasted_iota(jnp.int32, (NRB, NRB), 0)
    c64 = lax.broadcasted_iota(jnp.int32, (NRB, NRB), 1)
    Ls = (c64 < r64).astype(jnp.float32)

    def ranks(m):
        incl = lax.dot_general(m, U, (((1,), (0,)), ((), ())),
                               preferred_element_type=jnp.float32)
        rowsum = jnp.sum(m, axis=1, keepdims=True)
        rowpre = lax.dot_general(Ls, rowsum, (((1,), (0,)), ((), ())),
                                 preferred_element_type=jnp.float32)
        return rowpre + incl - m, jnp.sum(rowsum)

    dest0 = jnp.zeros((NRB, RBLK), jnp.float32)
    dest1 = jnp.zeros((NRB, RBLK), jnp.float32)
    off = jnp.float32(0.0)
    ends = []
    for e in range(E):
        m0 = (i0 == e).astype(jnp.float32)
        m1 = (i1 == e).astype(jnp.float32)
        r0, c0 = ranks(m0)
        r1, c1 = ranks(m1)
        dest0 = jnp.where(m0 > 0, off + r0, dest0)
        dest1 = jnp.where(m1 > 0, off + c0 + r1, dest1)
        cpad = jnp.ceil((c0 + c1) / TM) * TM
        off = off + cpad
        ends.append(off)
    d0_ref[...] = dest0.astype(jnp.int32)
    d1_ref[...] = dest1.astype(jnp.int32)
    g_slot = lax.broadcasted_iota(jnp.int32, (E, 128), 1).astype(
        jnp.float32) * TM
    acc = jnp.zeros((E, 128), jnp.int32)
    for e in range(E):
        acc = acc + (g_slot >= ends[e]).astype(jnp.int32)
    eot_ref[...] = jnp.minimum(acc, E - 1)
    lvec = load_ref[...][:, 0:1]
    lmean = jnp.sum(lvec) / E
    lvar = jnp.sum((lvec - lmean) ** 2) / (E - 1)
    cv = lvar / (lmean * lmean + 1e-10)
    vq_sum = vqs_ref[0, 0]
    mem = mem_ref[...]
    memn = mem / jnp.maximum(
        jnp.sqrt(jnp.sum(mem * mem, axis=1, keepdims=True)), 1e-12)
    gram = lax.dot_general(memn, memn, (((1,), (1,)), ((), ())),
                           preferred_element_type=jnp.float32)
    sim = jnp.sum(gram) / (E * E)
    aux = 0.05 * (-vq_sum / N) + 0.01 * sim + 0.01 * cv
    aux_ref[...] = jnp.full((E, 128), aux)


def _dispatch(i0, i1, load, vqs, routing_memory):
    return pl.pallas_call(
        _dispatch_body,
        out_shape=[
            jax.ShapeDtypeStruct((NRB, RBLK), jnp.int32),
            jax.ShapeDtypeStruct((NRB, RBLK), jnp.int32),
            jax.ShapeDtypeStruct((E, 128), jnp.int32),
            jax.ShapeDtypeStruct((E, 128), jnp.float32),
        ],
    )(i0, i1, load, vqs, routing_memory)


def _scatter_gather_body(d0_hbm, d1_hbm, w0_hbm, w1_hbm, x_hbm,
                         ws_hbm, xg_hbm,
                         d0v, d1v, w0v, w1v, stv, wsv, rowbuf, sem):
    wid = lax.axis_index("c") * 16 + lax.axis_index("s")
    lo = wid * SLOTS_W

    @pl.loop(0, SLOTS_WP // 16)
    def _zero(j):
        stv[pl.ds(j * 16, 16)] = jnp.zeros((16,), jnp.int32)
        wsv[pl.ds(j * 16, 16)] = jnp.zeros((16,), jnp.float32)

    pltpu.sync_copy(d0_hbm, d0v)
    pltpu.sync_copy(d1_hbm, d1v)
    pltpu.sync_copy(w0_hbm, w0v)
    pltpu.sync_copy(w1_hbm, w1v)

    def scan(dv, wv):
        @pl.loop(0, N // 16)
        def _scan(j):
            p16 = j * 16
            dvec = dv[pl.ds(p16, 16)]
            offv = dvec - lo
            m = (offv >= 0) & (offv < SLOTS_W)
            offc = jnp.where(m, offv, 0)
            tok = lax.iota(jnp.int32, 16) + p16
            plsc.store_scatter(stv, [offc], tok, mask=m)
            plsc.store_scatter(wsv, [offc], wv[pl.ds(p16, 16)], mask=m)

    scan(d0v, w0v)
    scan(d1v, w1v)
    pltpu.sync_copy(wsv.at[pl.ds(0, SLOTS_W)], ws_hbm.at[pl.ds(lo, SLOTS_W)])

    @pl.loop(0, SLOTS_W // GROWS)
    def _gather(c):
        idx = stv.at[pl.ds(c * GROWS, GROWS)]
        pltpu.async_copy(x_hbm.at[idx], rowbuf, sem).wait()
        pltpu.sync_copy(rowbuf, xg_hbm.at[pl.ds(lo + c * GROWS, GROWS)])


def _scatter_gather(d0, d1, w0, w1, x):
    mesh = plsc.VectorSubcoreMesh(core_axis_name="c", subcore_axis_name="s",
                                  num_cores=2, num_subcores=16)
    f = pl.kernel(
        _scatter_gather_body,
        compiler_params=pltpu.CompilerParams(needs_layout_passes=False),
        out_type=(
            jax.ShapeDtypeStruct((SPAD,), jnp.float32),
            jax.ShapeDtypeStruct((SPAD, D // 2), jnp.int32),
        ),
        mesh=mesh,
        scratch_types=[
            pltpu.VMEM((N,), jnp.int32),
            pltpu.VMEM((N,), jnp.int32),
            pltpu.VMEM((N,), jnp.float32),
            pltpu.VMEM((N,), jnp.float32),
            pltpu.VMEM((SLOTS_WP,), jnp.int32),
            pltpu.VMEM((SLOTS_WP,), jnp.float32),
            pltpu.VMEM((GROWS, D // 2), jnp.int32),
            pltpu.SemaphoreType.DMA,
        ],
    )
    return f(d0, d1, w0, w1, x)


def _gmm_body(eot_s, att_s, xg_ref, w_ref, b_ref, ws_ref, y_ref):
    g = pl.program_id(0)
    e = eot_s[g]
    xg = xg_ref[...]
    w = w_ref[...][0]
    mm = lax.dot_general(xg, w, (((1,), (1,)), ((), ())),
                         preferred_element_type=jnp.float32)
    wcol = ws_ref[...].reshape(TM, 1)
    y_ref[...] = wcol * (att_s[e] * mm + b_ref[...].reshape(1, D))


def _gmm(eot_s, att, xg, W, b, ws_r):
    grid_spec = pltpu.PrefetchScalarGridSpec(
        num_scalar_prefetch=2,
        grid=(NT,),
        in_specs=[
            pl.BlockSpec((TM, D), lambda g, s0, s1: (g, 0)),
            pl.BlockSpec((1, D, D), lambda g, s0, s1: (s0[g], 0, 0)),
            pl.BlockSpec((1, 1, D), lambda g, s0, s1: (s0[g], 0, 0)),
            pl.BlockSpec((1, TM, 1), lambda g, s0, s1: (g, 0, 0)),
        ],
        out_specs=pl.BlockSpec((TM, D), lambda g, s0, s1: (g, 0)),
    )
    return pl.pallas_call(
        _gmm_body,
        grid_spec=grid_spec,
        out_shape=jax.ShapeDtypeStruct((SPAD, D), jnp.float32),
    )(eot_s, att, xg, W, b.reshape(E, 1, D), ws_r)


def _combine_body(y_hbm, d0_hbm, d1_hbm, out_hbm, p0v, p1v, bufa, bufb,
                  sema, semb):
    wid = lax.axis_index("c") * 16 + lax.axis_index("s")
    base = wid * TOK_W
    pltpu.sync_copy(d0_hbm.at[pl.ds(base, TOK_W)], p0v)
    pltpu.sync_copy(d1_hbm.at[pl.ds(base, TOK_W)], p1v)

    @pl.loop(0, TOK_W // CROWS)
    def _chunk(c):
        cp = pltpu.async_copy(
            y_hbm.at[p0v.at[pl.ds(c * CROWS, CROWS)]], bufa, sema)
        cq = pltpu.async_copy(
            y_hbm.at[p1v.at[pl.ds(c * CROWS, CROWS)]], bufb, semb)
        cp.wait()
        cq.wait()

        @pl.loop(0, CROWS)
        def _row(r):
            @pl.loop(0, D // 16)
            def _col(j):
                bufa[r, pl.ds(j * 16, 16)] = (
                    bufa[r, pl.ds(j * 16, 16)] + bufb[r, pl.ds(j * 16, 16)])

        pltpu.sync_copy(bufa, out_hbm.at[pl.ds(base + c * CROWS, CROWS)])


def _combine(y, d0, d1):
    mesh = plsc.VectorSubcoreMesh(core_axis_name="c", subcore_axis_name="s",
                                  num_cores=2, num_subcores=16)
    f = pl.kernel(
        _combine_body,
        compiler_params=pltpu.CompilerParams(needs_layout_passes=False),
        out_type=jax.ShapeDtypeStruct((N, D), jnp.float32),
        mesh=mesh,
        scratch_types=[
            pltpu.VMEM((TOK_W,), jnp.int32),
            pltpu.VMEM((TOK_W,), jnp.int32),
            pltpu.VMEM((CROWS, D), jnp.float32),
            pltpu.VMEM((CROWS, D), jnp.float32),
            pltpu.SemaphoreType.DMA,
            pltpu.SemaphoreType.DMA,
        ],
    )
    return f(y, d0, d1)


@jax.jit
def kernel(x, routing_memory, W, b, temperature, const_attention):
    i0, i1, w0s, w1s, load, vqs = _router(x, routing_memory, temperature)
    d0, d1, eot, auxo = _dispatch(i0, i1, load, vqs, routing_memory)
    d0f = d0.reshape(N)
    d1f = d1.reshape(N)
    w0f = w0s.reshape(N)
    w1f = w1s.reshape(N)
    xi = lax.bitcast_convert_type(
        x.astype(jnp.bfloat16).reshape(N, D // 2, 2), jnp.int32)
    ws, xgi = _scatter_gather(d0f, d1f, w0f, w1f, xi)
    xg = lax.bitcast_convert_type(xgi, jnp.bfloat16).reshape(SPAD, D)
    eot_s = eot[0, :NT]
    att = jnp.exp(jnp.minimum(const_attention, _CLAMP))
    ws_r = ws.reshape(NT, TM, 1)
    y = _gmm(eot_s, att, xg, W.astype(jnp.bfloat16), b, ws_r)
    final = _combine(y, d0f, d1f)
    return final, auxo[0, 0]


# speedup vs baseline: 2.4393x; 2.4393x over previous
import functools

import jax
import jax.numpy as jnp
from jax import lax
from jax.experimental import pallas as pl
from jax.experimental.pallas import tpu as pltpu
from jax.experimental.pallas import tpu_sc as plsc

N = 8192
D = 2048
E = 8
K = 2
TM = 256
NT = (N * K) // TM + E
SPAD = NT * TM
NW = 32
SLOTS_W = SPAD // NW
SLOTS_WP = 640
TOK_W = N // NW
GROWS = 16
CROWS = 8
RBLK = 128
NRB = N // RBLK

_CLAMP = float(jnp.log(jnp.asarray(100.0, dtype=jnp.float32)))


def _router_body(temp_ref, x_ref, mem_ref, i0_ref, i1_ref, w0_ref, w1_ref,
                 load_ref, vqs_ref):
    i = pl.program_id(0)
    x = x_ref[...]
    mem = mem_ref[...]
    memn = mem / jnp.maximum(
        jnp.sqrt(jnp.sum(mem * mem, axis=1, keepdims=True)), 1e-12)
    xn = x / jnp.maximum(
        jnp.sqrt(jnp.sum(x * x, axis=1, keepdims=True)), 1e-12)
    dists = lax.dot_general(memn, xn, (((1,), (1,)), ((), ())),
                            preferred_element_type=jnp.float32)
    e_iota = lax.broadcasted_iota(jnp.int32, (E, RBLK), 0)
    v0 = jnp.max(dists, axis=0, keepdims=True)
    i0 = jnp.min(jnp.where(dists >= v0, e_iota, E), axis=0, keepdims=True)
    masked = jnp.where(e_iota == i0, -2.0, dists)
    v1 = jnp.max(masked, axis=0, keepdims=True)
    i1 = jnp.min(jnp.where(masked >= v1, e_iota, E), axis=0, keepdims=True)
    e1 = jnp.exp(v1 - v0)
    w0 = 1.0 / (1.0 + e1)
    w1 = e1 / (1.0 + e1)
    scale = jnp.exp(jnp.minimum(temp_ref[0], _CLAMP))
    i0_ref[...] = i0.reshape(1, 1, RBLK)
    i1_ref[...] = i1.reshape(1, 1, RBLK)
    w0_ref[...] = (scale * w0).reshape(1, 1, RBLK)
    w1_ref[...] = (scale * w1).reshape(1, 1, RBLK)
    aw = (jnp.where(e_iota == i0, w0, 0.0)
          + jnp.where(e_iota == i1, w1, 0.0))
    load_part = jnp.sum(aw, axis=1, keepdims=True)
    vq_part = jnp.sum(w0 * v0 + w1 * v1)

    @pl.when(i == 0)
    def _():
        load_ref[...] = jnp.zeros((E, 128), jnp.float32)
        vqs_ref[...] = jnp.zeros((E, 128), jnp.float32)

    load_ref[...] += jnp.broadcast_to(load_part, (E, 128))
    vqs_ref[...] += jnp.full((E, 128), vq_part)


def _router(x, routing_memory, temperature):
    return pl.pallas_call(
        _router_body,
        grid=(NRB,),
        in_specs=[
            pl.BlockSpec(memory_space=pltpu.SMEM),
            pl.BlockSpec((RBLK, D), lambda i: (i, 0)),
            pl.BlockSpec((E, D), lambda i: (0, 0)),
        ],
        out_specs=[
            pl.BlockSpec((1, 1, RBLK), lambda i: (i, 0, 0)),
            pl.BlockSpec((1, 1, RBLK), lambda i: (i, 0, 0)),
            pl.BlockSpec((1, 1, RBLK), lambda i: (i, 0, 0)),
            pl.BlockSpec((1, 1, RBLK), lambda i: (i, 0, 0)),
            pl.BlockSpec((E, 128), lambda i: (0, 0)),
            pl.BlockSpec((E, 128), lambda i: (0, 0)),
        ],
        out_shape=[
            jax.ShapeDtypeStruct((NRB, 1, RBLK), jnp.int32),
            jax.ShapeDtypeStruct((NRB, 1, RBLK), jnp.int32),
            jax.ShapeDtypeStruct((NRB, 1, RBLK), jnp.float32),
            jax.ShapeDtypeStruct((NRB, 1, RBLK), jnp.float32),
            jax.ShapeDtypeStruct((E, 128), jnp.float32),
            jax.ShapeDtypeStruct((E, 128), jnp.float32),
        ],
    )(temperature, x, routing_memory)


def _dispatch_body(i0_ref, i1_ref, load_ref, vqs_ref, mem_ref,
                   d0_ref, d1_ref, eot_ref, aux_ref):
    i0 = i0_ref[...].reshape(NRB, RBLK)
    i1 = i1_ref[...].reshape(NRB, RBLK)
    r128 = lax.broadcasted_iota(jnp.int32, (RBLK, RBLK), 0)
    c128 = lax.broadcasted_iota(jnp.int32, (RBLK, RBLK), 1)
    U = (r128 <= c128).astype(jnp.float32)
    r64 = lax.broadcasted_iota(jnp.int32, (NRB, NRB), 0)
    c64 = lax.broadcasted_iota(jnp.int32, (NRB, NRB), 1)
    Ls = (c64 < r64).astype(jnp.float32)

    def ranks(m):
        incl = lax.dot_general(m, U, (((1,), (0,)), ((), ())),
                               preferred_element_type=jnp.float32)
        rowsum = jnp.sum(m, axis=1, keepdims=True)
        rowpre = lax.dot_general(Ls, rowsum, (((1,), (0,)), ((), ())),
                                 preferred_element_type=jnp.float32)
        return rowpre + incl - m, jnp.sum(rowsum)

    dest0 = jnp.zeros((NRB, RBLK), jnp.float32)
    dest1 = jnp.zeros((NRB, RBLK), jnp.float32)
    off = jnp.float32(0.0)
    ends = []
    for e in range(E):
        m0 = (i0 == e).astype(jnp.float32)
        m1 = (i1 == e).astype(jnp.float32)
        r0, c0 = ranks(m0)
        r1, c1 = ranks(m1)
        dest0 = jnp.where(m0 > 0, off + r0, dest0)
        dest1 = jnp.where(m1 > 0, off + c0 + r1, dest1)
        cpad = jnp.ceil((c0 + c1) / TM) * TM
        off = off + cpad
        ends.append(off)
    d0_ref[...] = dest0.astype(jnp.int32)
    d1_ref[...] = dest1.astype(jnp.int32)
    g_slot = lax.broadcasted_iota(jnp.int32, (E, 128), 1).astype(
        jnp.float32) * TM
    acc = jnp.zeros((E, 128), jnp.int32)
    for e in range(E):
        acc = acc + (g_slot >= ends[e]).astype(jnp.int32)
    eot_ref[...] = jnp.minimum(acc, E - 1)
    lvec = load_ref[...][:, 0:1]
    lmean = jnp.sum(lvec) / E
    lvar = jnp.sum((lvec - lmean) ** 2) / (E - 1)
    cv = lvar / (lmean * lmean + 1e-10)
    vq_sum = vqs_ref[0, 0]
    mem = mem_ref[...]
    memn = mem / jnp.maximum(
        jnp.sqrt(jnp.sum(mem * mem, axis=1, keepdims=True)), 1e-12)
    gram = lax.dot_general(memn, memn, (((1,), (1,)), ((), ())),
                           preferred_element_type=jnp.float32)
    sim = jnp.sum(gram) / (E * E)
    aux = 0.05 * (-vq_sum / N) + 0.01 * sim + 0.01 * cv
    aux_ref[...] = jnp.full((E, 128), aux)


def _dispatch(i0, i1, load, vqs, routing_memory):
    return pl.pallas_call(
        _dispatch_body,
        out_shape=[
            jax.ShapeDtypeStruct((NRB, RBLK), jnp.int32),
            jax.ShapeDtypeStruct((NRB, RBLK), jnp.int32),
            jax.ShapeDtypeStruct((E, 128), jnp.int32),
            jax.ShapeDtypeStruct((E, 128), jnp.float32),
        ],
    )(i0, i1, load, vqs, routing_memory)


def _scatter_gather_body(d0_hbm, d1_hbm, w0_hbm, w1_hbm, x_hbm,
                         ws_hbm, xg_hbm,
                         d0v, d1v, w0v, w1v, stv, wsv,
                         rowbuf0, rowbuf1, sem0, sem1):
    wid = lax.axis_index("c") * 16 + lax.axis_index("s")
    lo = wid * SLOTS_W

    @pl.loop(0, SLOTS_WP // 16)
    def _zero(j):
        stv[pl.ds(j * 16, 16)] = jnp.zeros((16,), jnp.int32)
        wsv[pl.ds(j * 16, 16)] = jnp.zeros((16,), jnp.float32)

    pltpu.sync_copy(d0_hbm, d0v)
    pltpu.sync_copy(d1_hbm, d1v)
    pltpu.sync_copy(w0_hbm, w0v)
    pltpu.sync_copy(w1_hbm, w1v)

    def scan(dv, wv):
        @pl.loop(0, N // 16)
        def _scan(j):
            p16 = j * 16
            dvec = dv[pl.ds(p16, 16)]
            offv = dvec - lo
            m = (offv >= 0) & (offv < SLOTS_W)
            offc = jnp.where(m, offv, 0)
            tok = lax.iota(jnp.int32, 16) + p16
            plsc.store_scatter(stv, [offc], tok, mask=m)
            plsc.store_scatter(wsv, [offc], wv[pl.ds(p16, 16)], mask=m)

    scan(d0v, w0v)
    scan(d1v, w1v)
    pltpu.sync_copy(wsv.at[pl.ds(0, SLOTS_W)], ws_hbm.at[pl.ds(lo, SLOTS_W)])

    def _ig(c, rb, sm):
        pltpu.async_copy(x_hbm.at[stv.at[pl.ds(c * GROWS, GROWS)]], rb, sm)

    def _igw(rb, sm):
        pltpu.make_async_copy(
            x_hbm.at[stv.at[pl.ds(0, GROWS)]], rb, sm).wait()

    def _st(c, rb):
        pltpu.sync_copy(rb, xg_hbm.at[pl.ds(lo + c * GROWS, GROWS)])

    nh = SLOTS_W // GROWS // 2
    _ig(0, rowbuf0, sem0)

    @pl.loop(0, nh)
    def _gather(h):
        c0 = h * 2
        _igw(rowbuf0, sem0)
        _ig(c0 + 1, rowbuf1, sem1)
        _st(c0, rowbuf0)
        _igw(rowbuf1, sem1)

        @pl.when(h < nh - 1)
        def _():
            _ig(c0 + 2, rowbuf0, sem0)

        _st(c0 + 1, rowbuf1)


def _scatter_gather(d0, d1, w0, w1, x):
    mesh = plsc.VectorSubcoreMesh(core_axis_name="c", subcore_axis_name="s",
                                  num_cores=2, num_subcores=16)
    f = pl.kernel(
        _scatter_gather_body,
        compiler_params=pltpu.CompilerParams(needs_layout_passes=False),
        out_type=(
            jax.ShapeDtypeStruct((SPAD,), jnp.float32),
            jax.ShapeDtypeStruct((SPAD, D), jnp.float32),
        ),
        mesh=mesh,
        scratch_types=[
            pltpu.VMEM((N,), jnp.int32),
            pltpu.VMEM((N,), jnp.int32),
            pltpu.VMEM((N,), jnp.float32),
            pltpu.VMEM((N,), jnp.float32),
            pltpu.VMEM((SLOTS_WP,), jnp.int32),
            pltpu.VMEM((SLOTS_WP,), jnp.float32),
            pltpu.VMEM((GROWS, D), jnp.float32),
            pltpu.VMEM((GROWS, D), jnp.float32),
            pltpu.SemaphoreType.DMA,
            pltpu.SemaphoreType.DMA,
        ],
    )
    return f(d0, d1, w0, w1, x)


def _gmm_body(eot_s, att_s, xg_ref, w_ref, b_ref, ws_ref, y_ref):
    g = pl.program_id(0)
    e = eot_s[g]
    xg = xg_ref[...]
    w = w_ref[...][0]
    mm = lax.dot_general(xg, w, (((1,), (1,)), ((), ())),
                         preferred_element_type=jnp.float32)
    wcol = ws_ref[...].reshape(TM, 1)
    y_ref[...] = wcol * (att_s[e] * mm + b_ref[...].reshape(1, D))


def _gmm(eot_s, att, xg, W, b, ws_r):
    grid_spec = pltpu.PrefetchScalarGridSpec(
        num_scalar_prefetch=2,
        grid=(NT,),
        in_specs=[
            pl.BlockSpec((TM, D), lambda g, s0, s1: (g, 0)),
            pl.BlockSpec((1, D, D), lambda g, s0, s1: (s0[g], 0, 0)),
            pl.BlockSpec((1, 1, D), lambda g, s0, s1: (s0[g], 0, 0)),
            pl.BlockSpec((1, TM, 1), lambda g, s0, s1: (g, 0, 0)),
        ],
        out_specs=pl.BlockSpec((TM, D), lambda g, s0, s1: (g, 0)),
    )
    return pl.pallas_call(
        _gmm_body,
        grid_spec=grid_spec,
        out_shape=jax.ShapeDtypeStruct((SPAD, D), jnp.float32),
    )(eot_s, att, xg, W, b.reshape(E, 1, D), ws_r)


def _combine_body(y_hbm, d0_hbm, d1_hbm, out_hbm, p0v, p1v,
                  a0, b0, a1, b1, sa0, sb0, sa1, sb1):
    wid = lax.axis_index("c") * 16 + lax.axis_index("s")
    base = wid * TOK_W
    pltpu.sync_copy(d0_hbm.at[pl.ds(base, TOK_W)], p0v)
    pltpu.sync_copy(d1_hbm.at[pl.ds(base, TOK_W)], p1v)

    def _cg(c, ba, bb, sa, sb):
        pltpu.async_copy(y_hbm.at[p0v.at[pl.ds(c * CROWS, CROWS)]], ba, sa)
        pltpu.async_copy(y_hbm.at[p1v.at[pl.ds(c * CROWS, CROWS)]], bb, sb)

    def _cgw(ba, bb, sa, sb):
        pltpu.make_async_copy(
            y_hbm.at[p0v.at[pl.ds(0, CROWS)]], ba, sa).wait()
        pltpu.make_async_copy(
            y_hbm.at[p1v.at[pl.ds(0, CROWS)]], bb, sb).wait()

    def _addstore(c, ba, bb):
        @pl.loop(0, CROWS)
        def _row(r):
            @pl.loop(0, D // 16)
            def _col(j):
                ba[r, pl.ds(j * 16, 16)] = (
                    ba[r, pl.ds(j * 16, 16)] + bb[r, pl.ds(j * 16, 16)])

        pltpu.sync_copy(ba, out_hbm.at[pl.ds(base + c * CROWS, CROWS)])

    nh = TOK_W // CROWS // 2
    _cg(0, a0, b0, sa0, sb0)

    @pl.loop(0, nh)
    def _chunk(h):
        c0 = h * 2
        _cgw(a0, b0, sa0, sb0)
        _cg(c0 + 1, a1, b1, sa1, sb1)
        _addstore(c0, a0, b0)
        _cgw(a1, b1, sa1, sb1)

        @pl.when(h < nh - 1)
        def _():
            _cg(c0 + 2, a0, b0, sa0, sb0)

        _addstore(c0 + 1, a1, b1)


def _combine(y, d0, d1):
    mesh = plsc.VectorSubcoreMesh(core_axis_name="c", subcore_axis_name="s",
                                  num_cores=2, num_subcores=16)
    f = pl.kernel(
        _combine_body,
        compiler_params=pltpu.CompilerParams(needs_layout_passes=False),
        out_type=jax.ShapeDtypeStruct((N, D), jnp.float32),
        mesh=mesh,
        scratch_types=[
            pltpu.VMEM((TOK_W,), jnp.int32),
            pltpu.VMEM((TOK_W,), jnp.int32),
            pltpu.VMEM((CROWS, D), jnp.float32),
            pltpu.VMEM((CROWS, D), jnp.float32),
            pltpu.VMEM((CROWS, D), jnp.float32),
            pltpu.VMEM((CROWS, D), jnp.float32),
            pltpu.SemaphoreType.DMA,
            pltpu.SemaphoreType.DMA,
            pltpu.SemaphoreType.DMA,
            pltpu.SemaphoreType.DMA,
        ],
    )
    return f(y, d0, d1)


@jax.jit
def kernel(x, routing_memory, W, b, temperature, const_attention):
    i0, i1, w0s, w1s, load, vqs = _router(x, routing_memory, temperature)
    d0, d1, eot, auxo = _dispatch(i0, i1, load, vqs, routing_memory)
    d0f = d0.reshape(N)
    d1f = d1.reshape(N)
    w0f = w0s.reshape(N)
    w1f = w1s.reshape(N)
    ws, xg = _scatter_gather(d0f, d1f, w0f, w1f, x)
    eot_s = eot[0, :NT]
    att = jnp.exp(jnp.minimum(const_attention, _CLAMP))
    ws_r = ws.reshape(NT, TM, 1)
    y = _gmm(eot_s, att, xg, W, b, ws_r)
    final = _combine(y, d0f, d1f)
    return final, auxo[0, 0]
